# Initial kernel scaffold; baseline (speedup 1.0000x reference)
import jax
import jax.numpy as jnp
from jax.experimental import pallas as pl


def kernel(input_ids, table):
    raise NotImplementedError("write your pallas kernel here")


# speedup vs baseline: 1.0937x; 1.0937x over previous
import functools

import jax
import jax.numpy as jnp
from jax import lax
from jax.experimental import pallas as pl
from jax.experimental.pallas import tpu as pltpu
from jax.experimental.pallas import tpu_sc as plsc

EMBED = 32
NC = 2
NS = 16
NW = NC * NS

BATCH = 16384
SEQ = 50
TOTAL = BATCH * SEQ
B_PER_W = TOTAL // NW
CHUNK = 1024
N_CHUNK = B_PER_W // CHUNK


def _make_gather():
    mesh = plsc.VectorSubcoreMesh(core_axis_name="c", subcore_axis_name="s")

    @functools.partial(
        pl.kernel,
        mesh=mesh,
        out_type=jax.ShapeDtypeStruct((TOTAL, EMBED), jnp.float32),
        scratch_types=[
            pltpu.VMEM((CHUNK,), jnp.int32),
            pltpu.VMEM((CHUNK, EMBED), jnp.float32),
            pltpu.SemaphoreType.DMA,
        ],
        compiler_params=pltpu.CompilerParams(use_tc_tiling_on_sc=False),
    )
    def k(ids_hbm, table_hbm, out_hbm, idx_v, rows_v, sem):
        wid = lax.axis_index("s") * NC + lax.axis_index("c")
        base = wid * B_PER_W

        def body(i, carry):
            off = base + i * CHUNK
            pltpu.sync_copy(ids_hbm.at[pl.ds(off, CHUNK)], idx_v)
            pltpu.async_copy(table_hbm.at[idx_v], rows_v, sem).wait()
            pltpu.sync_copy(rows_v, out_hbm.at[pl.ds(off, CHUNK)])
            return carry

        lax.fori_loop(0, N_CHUNK, body, 0)

    return k


_gather = _make_gather()


def kernel(input_ids, table):
    flat_ids = input_ids.reshape(TOTAL)
    out = _gather(flat_ids, table)
    return out.reshape(BATCH, SEQ, EMBED)


# speedup vs baseline: 1.1073x; 1.0125x over previous
import functools

import jax
import jax.numpy as jnp
from jax import lax
from jax.experimental import pallas as pl
from jax.experimental.pallas import tpu as pltpu
from jax.experimental.pallas import tpu_sc as plsc

EMBED = 32
NC = 2
NS = 16
NW = NC * NS

BATCH = 16384
SEQ = 50
TOTAL = BATCH * SEQ
B_PER_W = TOTAL // NW
CHUNK = 1280
N_CHUNK = B_PER_W // CHUNK
N_PAIR = N_CHUNK // 2


def _make_gather():
    mesh = plsc.VectorSubcoreMesh(core_axis_name="c", subcore_axis_name="s")

    @functools.partial(
        pl.kernel,
        mesh=mesh,
        out_type=jax.ShapeDtypeStruct((TOTAL, EMBED), jnp.float32),
        scratch_types=[
            pltpu.VMEM((CHUNK,), jnp.int32),
            pltpu.VMEM((CHUNK,), jnp.int32),
            pltpu.VMEM((CHUNK, EMBED), jnp.float32),
            pltpu.VMEM((CHUNK, EMBED), jnp.float32),
            pltpu.SemaphoreType.DMA,
            pltpu.SemaphoreType.DMA,
            pltpu.SemaphoreType.DMA,
            pltpu.SemaphoreType.DMA,
        ],
        compiler_params=pltpu.CompilerParams(use_tc_tiling_on_sc=False),
    )
    def k(ids_hbm, table_hbm, out_hbm, idx0, idx1, rows0, rows1,
          gs0, gs1, ss0, ss1):
        wid = lax.axis_index("s") * NC + lax.axis_index("c")
        base = wid * B_PER_W
        idx = (idx0, idx1)
        rows = (rows0, rows1)
        gs = (gs0, gs1)
        ss = (ss0, ss1)

        def body(g, carry):
            for b in (0, 1):
                i = 2 * g + b
                off = base + i * CHUNK
                p = 1 - b

                @pl.when(g > 0)
                def _wait_prev_store():
                    pltpu.make_async_copy(
                        rows[b],
                        out_hbm.at[pl.ds(off - 2 * CHUNK, CHUNK)],
                        ss[b],
                    ).wait()

                pltpu.sync_copy(ids_hbm.at[pl.ds(off, CHUNK)], idx[b])
                pltpu.async_copy(table_hbm.at[idx[b]], rows[b], gs[b])

                def _retire():
                    pltpu.make_async_copy(
                        table_hbm.at[idx[p]], rows[p], gs[p]).wait()
                    pltpu.async_copy(
                        rows[p], out_hbm.at[pl.ds(off - CHUNK, CHUNK)], ss[p])

                if b == 0:
                    pl.when(g > 0)(_retire)
                else:
                    _retire()
            return carry

        lax.fori_loop(0, N_PAIR, body, 0)

        last = base + (N_CHUNK - 1) * CHUNK
        pltpu.make_async_copy(table_hbm.at[idx1], rows1, gs1).wait()
        pltpu.async_copy(rows1, out_hbm.at[pl.ds(last, CHUNK)], ss1)
        pltpu.make_async_copy(
            rows0, out_hbm.at[pl.ds(last - CHUNK, CHUNK)], ss0).wait()
        pltpu.make_async_copy(
            rows1, out_hbm.at[pl.ds(last, CHUNK)], ss1).wait()

    return k


_gather = _make_gather()


def kernel(input_ids, table):
    flat_ids = input_ids.reshape(TOTAL)
    out = _gather(flat_ids, table)
    return out.reshape(BATCH, SEQ, EMBED)


# speedup vs baseline: 1.7857x; 1.6126x over previous
import functools

import jax
import jax.numpy as jnp
from jax import lax
from jax.experimental import pallas as pl
from jax.experimental.pallas import tpu as pltpu
from jax.experimental.pallas import tpu_sc as plsc

EMBED = 32
NC = 2
NS = 16
NW = NC * NS

BATCH = 16384
SEQ = 50
TOTAL = BATCH * SEQ
BATCH_PER_W = BATCH // NW
NB = 16
ROWS = NB * SEQ
N_CHUNK = BATCH_PER_W // NB
N_PAIR = N_CHUNK // 2


def _make_gather():
    mesh = plsc.VectorSubcoreMesh(core_axis_name="c", subcore_axis_name="s")

    @functools.partial(
        pl.kernel,
        mesh=mesh,
        out_type=jax.ShapeDtypeStruct((BATCH, SEQ, EMBED), jnp.float32),
        scratch_types=[
            pltpu.VMEM((ROWS,), jnp.int32),
            pltpu.VMEM((ROWS,), jnp.int32),
            pltpu.VMEM((ROWS, EMBED), jnp.float32),
            pltpu.VMEM((ROWS, EMBED), jnp.float32),
            pltpu.SemaphoreType.DMA,
            pltpu.SemaphoreType.DMA,
            pltpu.SemaphoreType.DMA,
            pltpu.SemaphoreType.DMA,
        ],
        compiler_params=pltpu.CompilerParams(use_tc_tiling_on_sc=False),
    )
    def k(ids_hbm, table_hbm, out_hbm, idx0, idx1, rows0, rows1,
          gs0, gs1, ss0, ss1):
        wid = lax.axis_index("s") * NC + lax.axis_index("c")
        bbase = wid * BATCH_PER_W
        idx = (idx0, idx1)
        rows = (rows0, rows1)
        gs = (gs0, gs1)
        ss = (ss0, ss1)

        def fire_stores(p, b0):
            for j in range(NB):
                pltpu.async_copy(
                    rows[p].at[pl.ds(j * SEQ, SEQ)], out_hbm.at[b0 + j], ss[p])

        def drain_stores(p):
            for _ in range(NB):
                pltpu.make_async_copy(
                    rows[p].at[pl.ds(0, SEQ)], out_hbm.at[0], ss[p]).wait()

        def body(g, carry):
            for b in (0, 1):
                i = 2 * g + b
                b0 = bbase + i * NB
                p = 1 - b

                @pl.when(g > 0)
                def _wait_prev_stores():
                    drain_stores(b)

                pltpu.sync_copy(ids_hbm.at[pl.ds(b0 * SEQ, ROWS)], idx[b])
                pltpu.async_copy(table_hbm.at[idx[b]], rows[b], gs[b])

                def _retire():
                    pltpu.make_async_copy(
                        table_hbm.at[idx[p]], rows[p], gs[p]).wait()
                    fire_stores(p, b0 - NB)

                if b == 0:
                    pl.when(g > 0)(_retire)
                else:
                    _retire()
            return carry

        lax.fori_loop(0, N_PAIR, body, 0)

        last_b0 = bbase + (N_CHUNK - 1) * NB
        pltpu.make_async_copy(table_hbm.at[idx1], rows1, gs1).wait()
        fire_stores(1, last_b0)
        drain_stores(0)
        drain_stores(1)

    return k


_gather = _make_gather()


def kernel(input_ids, table):
    flat_ids = input_ids.reshape(TOTAL)
    return _gather(flat_ids, table)
